# baseline (speedup 1.0000x reference)
import functools

import jax
import jax.numpy as jnp
from jax import lax
from jax.experimental import pallas as pl
from jax.experimental.pallas import tpu as pltpu
from jax.experimental.pallas import tpu_sc as plsc

B = 64
W = 512
BAND = 32 * W
KS = 32

NUM_CORES = 2
NUM_SUBCORES = 16
NW = NUM_CORES * NUM_SUBCORES

_mesh = plsc.VectorSubcoreMesh(
    core_axis_name="c", subcore_axis_name="s",
    num_cores=NUM_CORES, num_subcores=NUM_SUBCORES)


def _make_sc(nb):
    tasks = nb * 16
    tpw = tasks // NW

    @functools.partial(
        pl.kernel,
        out_type=jax.ShapeDtypeStruct((nb, 16, BAND), jnp.float32),
        mesh=_mesh,
        compiler_params=pltpu.CompilerParams(
            use_tc_tiling_on_sc=False, needs_layout_passes=False),
        scratch_types=[
            pltpu.VMEM((BAND,), jnp.float32),
            pltpu.VMEM((BAND,), jnp.float32),
            pltpu.VMEM((BAND,), jnp.float32),
            pltpu.VMEM((BAND,), jnp.float32),
            pltpu.SemaphoreType.DMA,
            pltpu.SemaphoreType.DMA,
            pltpu.SemaphoreType.DMA,
            pltpu.SemaphoreType.DMA,
        ],
    )
    def _sc(in_hbm, out_hbm, in_a, in_b, out_a, out_b, si_a, si_b, so_a, so_b):
        cid = lax.axis_index("c")
        sid = lax.axis_index("s")
        wid = sid * NUM_CORES + cid

        lane = lax.iota(jnp.int32, 16)
        c_lane = lane % 4
        flatpat = (c_lane // 2) * (16 * W) + (c_lane % 2) * 16 + lane // 4
        pats = [flatpat + (32 * (r >> 2) + 4 * (r & 3)) for r in range(8)]

        ins = [in_a, in_b]
        outs = [out_a, out_b]
        isems = [si_a, si_b]
        osems = [so_a, so_b]

        def hbm_in(t):
            task = wid * tpw + t
            return in_hbm.at[task // 16, task % 16]

        def hbm_out(t):
            task = wid * tpw + t
            return out_hbm.at[task // 16, task % 16]

        in_desc = [None, None]
        out_desc = [None, None]
        in_desc[0] = pltpu.async_copy(hbm_in(0), ins[0], isems[0])
        for t in range(tpw):
            sl = t % 2
            if t + 1 < tpw:
                in_desc[1 - sl] = pltpu.async_copy(
                    hbm_in(t + 1), ins[1 - sl], isems[1 - sl])
            in_desc[sl].wait()
            if out_desc[sl] is not None:
                out_desc[sl].wait()
            ibuf = ins[sl]
            obuf = outs[sl]

            @plsc.parallel_loop(0, 1024, step=8, unroll=2)
            def _chunk(m):
                base = m * 8
                dst = m * 16
                for r in range(8):
                    vals = plsc.load_gather(ibuf, [pats[r] + base])
                    obuf[pl.ds(dst + r * 16, 16)] = vals

            out_desc[sl] = pltpu.async_copy(obuf, hbm_out(t), osems[sl])
        out_desc[0].wait()
        out_desc[1].wait()

    return _sc


def _tc_body(in_ref, out_ref):
    j = lax.broadcasted_iota(jnp.int32, (16, 128), 1)
    p = j // 4
    base_idx = 32 * (p // 16) + 16 * (j % 2) + (p % 16)
    cmask = (j % 4) < 2
    for ys in range(16):
        x = in_ref[0, 32 * ys:32 * ys + 32]
        for v in range(8):
            w = v // 2
            s0 = x[0:16, 128 * w:128 * w + 128]
            s1 = x[16:32, 128 * w:128 * w + 128]
            idx = base_idx + (v % 2) * 64
            g0 = jnp.take_along_axis(s0, idx, axis=1)
            g1 = jnp.take_along_axis(s1, idx, axis=1)
            out_ref[0, 16 * ys:16 * ys + 16, 128 * v:128 * v + 128] = (
                jnp.where(cmask, g0, g1))


def _make_tc(nb):
    return pl.pallas_call(
        _tc_body,
        out_shape=jax.ShapeDtypeStruct((nb, 256, 1024), jnp.float32),
        grid=(nb,),
        in_specs=[pl.BlockSpec((1, 512, W), lambda b: (b, 0, 0))],
        out_specs=pl.BlockSpec((1, 256, 1024), lambda b: (b, 0, 0)),
        compiler_params=pltpu.CompilerParams(
            dimension_semantics=("arbitrary",)),
    )


def kernel(image):
    img = jnp.reshape(image, (B, W, W))
    out_sc = _make_sc(KS)(jnp.reshape(img[:KS], (KS, 16, BAND)))
    out_tc = _make_tc(B - KS)(img[KS:])
    out = jnp.concatenate(
        [jnp.reshape(out_sc, (KS, 256, 1024)), out_tc], axis=0)
    return jnp.reshape(out, (B, 256, 256, 4))


# speedup vs baseline: 1.7232x; 1.7232x over previous
import functools

import jax
import jax.numpy as jnp
from jax import lax
from jax.experimental import pallas as pl
from jax.experimental.pallas import tpu as pltpu
from jax.experimental.pallas import tpu_sc as plsc

B = 64
W = 512
BAND = 32 * W
HALF = BAND // 2

NUM_CORES = 2
NUM_SUBCORES = 16
NW = NUM_CORES * NUM_SUBCORES
TASKS = B * 16
TPW = TASKS // NW

N_IN = 3
N_OUT = 2

_mesh = plsc.VectorSubcoreMesh(
    core_axis_name="c", subcore_axis_name="s",
    num_cores=NUM_CORES, num_subcores=NUM_SUBCORES)


@functools.partial(
    pl.kernel,
    out_type=jax.ShapeDtypeStruct((B, 16, BAND), jnp.float32),
    mesh=_mesh,
    compiler_params=pltpu.CompilerParams(
        use_tc_tiling_on_sc=False, needs_layout_passes=False),
    scratch_types=(
        [pltpu.VMEM((BAND,), jnp.float32)] * (N_IN + N_OUT)
        + [pltpu.SemaphoreType.DMA] * (N_IN + 2 * N_OUT)
    ),
)
def _unweave(in_hbm, out_hbm, *refs):
    ins = list(refs[:N_IN])
    outs = list(refs[N_IN:N_IN + N_OUT])
    isems = list(refs[N_IN + N_OUT:2 * N_IN + N_OUT])
    osems = [list(refs[2 * N_IN + N_OUT + 2 * s:2 * N_IN + N_OUT + 2 * s + 2])
             for s in range(N_OUT)]

    cid = lax.axis_index("c")
    sid = lax.axis_index("s")
    wid = sid * NUM_CORES + cid

    lane = lax.iota(jnp.int32, 16)
    c_lane = lane % 4
    flatpat = (c_lane // 2) * (16 * W) + (c_lane % 2) * 16 + lane // 4
    pats = [flatpat + (32 * (r >> 2) + 4 * (r & 3)) for r in range(8)]

    def hbm_in(t):
        task = wid * TPW + t
        return in_hbm.at[task // 16, task % 16]

    def hbm_out(t, h):
        task = wid * TPW + t
        return out_hbm.at[task // 16, task % 16, pl.ds(h * HALF, HALF)]

    in_desc = [None] * N_IN
    out_desc = [[None, None] for _ in range(N_OUT)]
    for u in range(min(N_IN - 1, TPW)):
        in_desc[u] = pltpu.async_copy(hbm_in(u), ins[u], isems[u])
    for t in range(TPW):
        isl = t % N_IN
        osl = t % N_OUT
        u = t + N_IN - 1
        if u < TPW:
            in_desc[u % N_IN] = pltpu.async_copy(
                hbm_in(u), ins[u % N_IN], isems[u % N_IN])
        in_desc[isl].wait()
        ibuf = ins[isl]
        obuf = outs[osl]
        for h in range(2):
            if out_desc[osl][h] is not None:
                out_desc[osl][h].wait()

            @plsc.parallel_loop(512 * h, 512 * (h + 1), step=8, unroll=1)
            def _chunk(m):
                base = m * 8
                dst = m * 16
                for r in range(8):
                    vals = plsc.load_gather(ibuf, [pats[r] + base])
                    obuf[pl.ds(dst + r * 16, 16)] = vals

            out_desc[osl][h] = pltpu.async_copy(
                obuf.at[pl.ds(h * HALF, HALF)], hbm_out(t, h), osems[osl][h])
    for ds in out_desc:
        for d in ds:
            if d is not None:
                d.wait()


def kernel(image):
    img = jnp.reshape(image, (B, 16, BAND))
    out = _unweave(img)
    return jnp.reshape(out, (B, 256, 256, 4))


# speedup vs baseline: 1.7753x; 1.0302x over previous
import functools

import jax
import jax.numpy as jnp
from jax import lax
from jax.experimental import pallas as pl
from jax.experimental.pallas import tpu as pltpu
from jax.experimental.pallas import tpu_sc as plsc

B = 64
W = 512
BAND = 32 * W
HALF = BAND // 2

NUM_CORES = 2
NUM_SUBCORES = 16
NW = NUM_CORES * NUM_SUBCORES
TASKS = B * 16
TPW = TASKS // NW

N_IN = 3
N_OUT = 2

_mesh = plsc.VectorSubcoreMesh(
    core_axis_name="c", subcore_axis_name="s",
    num_cores=NUM_CORES, num_subcores=NUM_SUBCORES)


@functools.partial(
    pl.kernel,
    out_type=jax.ShapeDtypeStruct((B, 16, BAND), jnp.float32),
    mesh=_mesh,
    compiler_params=pltpu.CompilerParams(
        use_tc_tiling_on_sc=False, needs_layout_passes=False),
    scratch_types=(
        [pltpu.VMEM((BAND,), jnp.float32)] * (N_IN + N_OUT)
        + [pltpu.SemaphoreType.DMA] * (N_IN + N_OUT)
    ),
)
def _unweave(in_hbm, out_hbm, *refs):
    ins = list(refs[:N_IN])
    outs = list(refs[N_IN:N_IN + N_OUT])
    isems = list(refs[N_IN + N_OUT:2 * N_IN + N_OUT])
    osems = list(refs[2 * N_IN + N_OUT:])

    cid = lax.axis_index("c")
    sid = lax.axis_index("s")
    wid = sid * NUM_CORES + cid

    lane = lax.iota(jnp.int32, 16)
    c_lane = lane % 4
    flatpat = (c_lane // 2) * (16 * W) + (c_lane % 2) * 16 + lane // 4
    pats = [flatpat + (32 * (r >> 2) + 4 * (r & 3)) for r in range(8)]

    def hbm_in(t):
        task = wid * TPW + t
        return in_hbm.at[task // 16, task % 16]

    def hbm_out(t):
        task = wid * TPW + t
        return out_hbm.at[task // 16, task % 16]

    in_desc = [None] * N_IN
    out_desc = [None] * N_OUT
    for u in range(min(N_IN - 1, TPW)):
        in_desc[u] = pltpu.async_copy(hbm_in(u), ins[u], isems[u])
    for t in range(TPW):
        isl = t % N_IN
        osl = t % N_OUT
        u = t + N_IN - 1
        if u < TPW:
            in_desc[u % N_IN] = pltpu.async_copy(
                hbm_in(u), ins[u % N_IN], isems[u % N_IN])
        in_desc[isl].wait()
        if out_desc[osl] is not None:
            out_desc[osl].wait()
        ibuf = ins[isl]
        obuf = outs[osl]

        @plsc.parallel_loop(0, 1024, step=1, unroll=8)
        def _chunk(m):
            off = (m >> 6) * W + (m & 3) * 4 + ((m >> 2) & 15) * 32
            vals = plsc.load_gather(ibuf, [flatpat + off])
            obuf[pl.ds(m * 16, 16)] = vals

        out_desc[osl] = pltpu.async_copy(obuf, hbm_out(t), osems[osl])
    for d in out_desc:
        if d is not None:
            d.wait()


def kernel(image):
    img = jnp.reshape(image, (B, 16, BAND))
    out = _unweave(img)
    return jnp.reshape(out, (B, 256, 256, 4))


# speedup vs baseline: 1.7844x; 1.0051x over previous
import functools

import jax
import jax.numpy as jnp
from jax import lax
from jax.experimental import pallas as pl
from jax.experimental.pallas import tpu as pltpu
from jax.experimental.pallas import tpu_sc as plsc

B = 64
W = 512
BAND = 32 * W
HALF = BAND // 2

NUM_CORES = 2
NUM_SUBCORES = 16
NW = NUM_CORES * NUM_SUBCORES
TASKS = B * 16
TPW = TASKS // NW

N_IN = 3
N_OUT = 2

_mesh = plsc.VectorSubcoreMesh(
    core_axis_name="c", subcore_axis_name="s",
    num_cores=NUM_CORES, num_subcores=NUM_SUBCORES)


@functools.partial(
    pl.kernel,
    out_type=jax.ShapeDtypeStruct((B, 16, BAND), jnp.float32),
    mesh=_mesh,
    compiler_params=pltpu.CompilerParams(
        use_tc_tiling_on_sc=False, needs_layout_passes=False),
    scratch_types=(
        [pltpu.VMEM((BAND,), jnp.float32)] * (N_IN + N_OUT)
        + [pltpu.SemaphoreType.DMA] * (N_IN + N_OUT)
    ),
)
def _unweave(in_hbm, out_hbm, *refs):
    ins = list(refs[:N_IN])
    outs = list(refs[N_IN:N_IN + N_OUT])
    isems = list(refs[N_IN + N_OUT:2 * N_IN + N_OUT])
    osems = list(refs[2 * N_IN + N_OUT:])

    cid = lax.axis_index("c")
    sid = lax.axis_index("s")
    wid = sid * NUM_CORES + cid

    lane = lax.iota(jnp.int32, 16)
    c_lane = lane % 4
    flatpat = (c_lane // 2) * (16 * W) + (c_lane % 2) * 16 + lane // 4
    pats = [flatpat + (32 * (r >> 2) + 4 * (r & 3)) for r in range(8)]

    def hbm_in(t):
        task = t * NW + wid
        return in_hbm.at[task // 16, task % 16]

    def hbm_out(t):
        task = t * NW + wid
        return out_hbm.at[task // 16, task % 16]

    in_desc = [None] * N_IN
    out_desc = [None] * N_OUT
    for u in range(min(N_IN - 1, TPW)):
        in_desc[u] = pltpu.async_copy(hbm_in(u), ins[u], isems[u])
    for t in range(TPW):
        isl = t % N_IN
        osl = t % N_OUT
        u = t + N_IN - 1
        if u < TPW:
            in_desc[u % N_IN] = pltpu.async_copy(
                hbm_in(u), ins[u % N_IN], isems[u % N_IN])
        in_desc[isl].wait()
        if out_desc[osl] is not None:
            out_desc[osl].wait()
        ibuf = ins[isl]
        obuf = outs[osl]

        @plsc.parallel_loop(0, 1024, step=1, unroll=8)
        def _chunk(m):
            off = (m >> 6) * W + (m & 3) * 4 + ((m >> 2) & 15) * 32
            vals = plsc.load_gather(ibuf, [flatpat + off])
            obuf[pl.ds(m * 16, 16)] = vals

        out_desc[osl] = pltpu.async_copy(obuf, hbm_out(t), osems[osl])
    for d in out_desc:
        if d is not None:
            d.wait()


def kernel(image):
    img = jnp.reshape(image, (B, 16, BAND))
    out = _unweave(img)
    return jnp.reshape(out, (B, 256, 256, 4))
